# baseline (speedup 1.0000x reference)
import functools

import jax
import jax.numpy as jnp
from jax import lax
from jax.experimental import pallas as pl
from jax.experimental.pallas import tpu as pltpu
from jax.experimental.pallas import tpu_sc as plsc

B = 16384
D = 64
H1 = 128
H2 = 64

_NC = 2
_NS = 16
_NW = _NC * _NS
_BPW = B // _NW
_CH = 128
_R = 256

_BT = 2048


def _gather_body(uid_hbm, mid_hbm, u2_hbm, m2_hbm, gu_out, gm_out,
                 uidx_v, midx_v, gu_v, gm_v, sem):
    wid = lax.axis_index("s") * _NC + lax.axis_index("c")
    base = wid * _BPW
    pltpu.sync_copy(uid_hbm.at[pl.ds(base, _BPW)], uidx_v)
    pltpu.sync_copy(mid_hbm.at[pl.ds(base, _BPW)], midx_v)

    def halve(i, carry):
        sl = pl.ds(i * 16, 16)
        uidx_v[sl] = lax.shift_right_logical(uidx_v[sl], 1)
        midx_v[sl] = lax.shift_right_logical(midx_v[sl], 1)
        return carry

    lax.fori_loop(0, _BPW // 16, halve, 0)

    for rnd in range(_BPW // _R):
        hs = []
        for ci in range(_R // _CH):
            sl_src = pl.ds(rnd * _R + ci * _CH, _CH)
            sl_dst = pl.ds(ci * _CH, _CH)
            hs.append(pltpu.async_copy(
                u2_hbm.at[uidx_v.at[sl_src]], gu_v.at[sl_dst], sem))
            hs.append(pltpu.async_copy(
                m2_hbm.at[midx_v.at[sl_src]], gm_v.at[sl_dst], sem))
        for h in hs:
            h.wait()
        pltpu.sync_copy(gu_v, gu_out.at[pl.ds(base + rnd * _R, _R)])
        pltpu.sync_copy(gm_v, gm_out.at[pl.ds(base + rnd * _R, _R)])


@functools.cache
def _make_gather():
    return pl.kernel(
        _gather_body,
        mesh=plsc.VectorSubcoreMesh(core_axis_name="c", subcore_axis_name="s"),
        out_type=[
            jax.ShapeDtypeStruct((B, 2 * D), jnp.float32),
            jax.ShapeDtypeStruct((B, 2 * D), jnp.float32),
        ],
        scratch_types=[
            pltpu.VMEM((_BPW,), jnp.int32),
            pltpu.VMEM((_BPW,), jnp.int32),
            pltpu.VMEM((_R, 2 * D), jnp.float32),
            pltpu.VMEM((_R, 2 * D), jnp.float32),
            pltpu.SemaphoreType.DMA,
        ],
        compiler_params=pltpu.CompilerParams(use_tc_tiling_on_sc=True),
    )


def _mlp_body(gu_ref, gm_ref, uid_ref, mid_ref, w1u_ref, w1m_ref, b1_ref,
              w2_ref, b2_ref, w3_ref, b3_ref, out_ref):
    pu = lax.bitwise_and(uid_ref[...], 1) == 1
    pm = lax.bitwise_and(mid_ref[...], 1) == 1
    u = jnp.where(pu, gu_ref[:, D:], gu_ref[:, :D])
    m = jnp.where(pm, gm_ref[:, D:], gm_ref[:, :D])
    h1 = jnp.dot(u, w1u_ref[...], preferred_element_type=jnp.float32)
    h1 = h1 + jnp.dot(m, w1m_ref[...], preferred_element_type=jnp.float32)
    h1 = jnp.maximum(h1 + b1_ref[...], 0.0)
    h2 = jnp.dot(h1, w2_ref[...], preferred_element_type=jnp.float32)
    h2 = jnp.maximum(h2 + b2_ref[...], 0.0)
    out_ref[...] = jnp.sum(h2 * w3_ref[...], axis=1, keepdims=True) + b3_ref[...]


@functools.cache
def _make_mlp():
    return pl.pallas_call(
        _mlp_body,
        grid=(B // _BT,),
        in_specs=[
            pl.BlockSpec((_BT, 2 * D), lambda i: (i, 0)),
            pl.BlockSpec((_BT, 2 * D), lambda i: (i, 0)),
            pl.BlockSpec((_BT, 1), lambda i: (i, 0)),
            pl.BlockSpec((_BT, 1), lambda i: (i, 0)),
            pl.BlockSpec((D, H1), lambda i: (0, 0)),
            pl.BlockSpec((D, H1), lambda i: (0, 0)),
            pl.BlockSpec((1, H1), lambda i: (0, 0)),
            pl.BlockSpec((H1, H2), lambda i: (0, 0)),
            pl.BlockSpec((1, H2), lambda i: (0, 0)),
            pl.BlockSpec((1, H2), lambda i: (0, 0)),
            pl.BlockSpec((1, 1), lambda i: (0, 0)),
        ],
        out_specs=pl.BlockSpec((_BT, 1), lambda i: (i, 0)),
        out_shape=jax.ShapeDtypeStruct((B, 1), jnp.float32),
        compiler_params=pltpu.CompilerParams(
            dimension_semantics=("arbitrary",),
        ),
    )


def kernel(user_id, movie_id, user_emb, movie_emb, W1, b1, W2, b2, W3, b3):
    uid = user_id.astype(jnp.int32)
    mid = movie_id.astype(jnp.int32)
    u2 = user_emb.reshape(-1, 2 * D)
    m2 = movie_emb.reshape(-1, 2 * D)
    gu, gm = _make_gather()(uid, mid, u2, m2)
    return _make_mlp()(
        gu, gm, uid.reshape(B, 1), mid.reshape(B, 1),
        W1[:, :D].T, W1[:, D:].T, b1.reshape(1, H1),
        W2.T, b2.reshape(1, H2),
        W3.reshape(1, H2), b3.reshape(1, 1),
    )


# speedup vs baseline: 1.9489x; 1.9489x over previous
import functools

import jax
import jax.numpy as jnp
from jax import lax
from jax.experimental import pallas as pl
from jax.experimental.pallas import tpu as pltpu
from jax.experimental.pallas import tpu_sc as plsc

B = 16384
D = 64
H1 = 128
H2 = 64

_NC = 2
_NS = 16
_NW = _NC * _NS
_BPW = B // _NW
_CH = 128
_NCH = _BPW // _CH

_BT = 2048


_W = 16
_R = 256


def _gather_body(uid_hbm, mid_hbm, uemb_hbm, memb_hbm, u_out, m_out,
                 uidx_v, midx_v, urows_v, mrows_v, sem):
    wid = lax.axis_index("s") * _NC + lax.axis_index("c")
    base = wid * _BPW
    pltpu.sync_copy(uid_hbm.at[pl.ds(base, _BPW)], uidx_v)
    pltpu.sync_copy(mid_hbm.at[pl.ds(base, _BPW)], midx_v)

    for rnd in range(_BPW // _R):
        def wave(w, carry):
            cu = uidx_v[pl.ds(rnd * _R + w * _W, _W)]
            cm = midx_v[pl.ds(rnd * _R + w * _W, _W)]
            hs = []
            for j in range(_W):
                rr = w * _W + j
                hs.append(pltpu.async_copy(
                    uemb_hbm.at[cu[j]], urows_v.at[rr, pl.ds(0, D)], sem))
                hs.append(pltpu.async_copy(
                    memb_hbm.at[cm[j]], mrows_v.at[rr, pl.ds(0, D)], sem))
            for h in hs:
                h.wait()
            return carry

        lax.fori_loop(0, _R // _W, wave, 0)
        pltpu.sync_copy(urows_v, u_out.at[pl.ds(base + rnd * _R, _R)])
        pltpu.sync_copy(mrows_v, m_out.at[pl.ds(base + rnd * _R, _R)])


@functools.cache
def _make_gather():
    return pl.kernel(
        _gather_body,
        mesh=plsc.VectorSubcoreMesh(core_axis_name="c", subcore_axis_name="s"),
        out_type=[
            jax.ShapeDtypeStruct((B, 2 * D), jnp.float32),
            jax.ShapeDtypeStruct((B, 2 * D), jnp.float32),
        ],
        scratch_types=[
            pltpu.VMEM((_BPW,), jnp.int32),
            pltpu.VMEM((_BPW,), jnp.int32),
            pltpu.VMEM((_R, 2 * D), jnp.float32),
            pltpu.VMEM((_R, 2 * D), jnp.float32),
            pltpu.SemaphoreType.DMA,
        ],
        compiler_params=pltpu.CompilerParams(use_tc_tiling_on_sc=True),
    )


_BTC = 8192


def _tr_body(x_ref, o_ref):
    o_ref[...] = x_ref[...].T


@functools.cache
def _make_tr(n):
    grid = (n + _BTC - 1) // _BTC
    return pl.pallas_call(
        _tr_body,
        grid=(grid,),
        in_specs=[pl.BlockSpec((D, _BTC), lambda i: (0, i))],
        out_specs=pl.BlockSpec((_BTC, D), lambda i: (i, 0)),
        out_shape=jax.ShapeDtypeStruct((n, D), jnp.float32),
        compiler_params=pltpu.CompilerParams(
            dimension_semantics=("arbitrary",),
        ),
    )


def _mlp_body(u_ref, m_ref, w1u_ref, w1m_ref, b1_ref, w2_ref, b2_ref,
              w3_ref, b3_ref, out_ref):
    h1 = jnp.dot(u_ref[:, :D], w1u_ref[...], preferred_element_type=jnp.float32)
    h1 = h1 + jnp.dot(m_ref[:, :D], w1m_ref[...], preferred_element_type=jnp.float32)
    h1 = jnp.maximum(h1 + b1_ref[...], 0.0)
    h2 = jnp.dot(h1, w2_ref[...], preferred_element_type=jnp.float32)
    h2 = jnp.maximum(h2 + b2_ref[...], 0.0)
    out_ref[...] = jnp.sum(h2 * w3_ref[...], axis=1, keepdims=True) + b3_ref[...]


@functools.cache
def _make_mlp():
    return pl.pallas_call(
        _mlp_body,
        grid=(B // _BT,),
        in_specs=[
            pl.BlockSpec((_BT, 2 * D), lambda i: (i, 0)),
            pl.BlockSpec((_BT, 2 * D), lambda i: (i, 0)),
            pl.BlockSpec((D, H1), lambda i: (0, 0)),
            pl.BlockSpec((D, H1), lambda i: (0, 0)),
            pl.BlockSpec((1, H1), lambda i: (0, 0)),
            pl.BlockSpec((H1, H2), lambda i: (0, 0)),
            pl.BlockSpec((1, H2), lambda i: (0, 0)),
            pl.BlockSpec((1, H2), lambda i: (0, 0)),
            pl.BlockSpec((1, 1), lambda i: (0, 0)),
        ],
        out_specs=pl.BlockSpec((_BT, 1), lambda i: (i, 0)),
        out_shape=jax.ShapeDtypeStruct((B, 1), jnp.float32),
        compiler_params=pltpu.CompilerParams(
            dimension_semantics=("arbitrary",),
        ),
    )


def kernel(user_id, movie_id, user_emb, movie_emb, W1, b1, W2, b2, W3, b3):
    ue_rm = _make_tr(user_emb.shape[0])(user_emb.T)
    me_rm = _make_tr(movie_emb.shape[0])(movie_emb.T)
    u, m = _make_gather()(user_id.astype(jnp.int32), movie_id.astype(jnp.int32),
                          ue_rm, me_rm)
    return _make_mlp()(
        u, m,
        W1[:, :D].T, W1[:, D:].T, b1.reshape(1, H1),
        W2.T, b2.reshape(1, H2),
        W3.reshape(1, H2), b3.reshape(1, 1),
    )


# speedup vs baseline: 2.2305x; 1.1445x over previous
import functools

import jax
import jax.numpy as jnp
from jax import lax
from jax.experimental import pallas as pl
from jax.experimental.pallas import tpu as pltpu
from jax.experimental.pallas import tpu_sc as plsc

B = 16384
D = 64
H1 = 128
H2 = 64

_NC = 2
_NS = 16
_NW = _NC * _NS
_BPW = B // _NW
_CH = 128
_NCH = _BPW // _CH

_BT = 2048


_W = 16
_R = 256


def _gather_body(uid_hbm, mid_hbm, u2_hbm, m2_hbm, gu_out, gm_out,
                 uidx_v, midx_v, gu_v, gm_v, sem):
    wid = lax.axis_index("s") * _NC + lax.axis_index("c")
    base = wid * _BPW
    pltpu.sync_copy(uid_hbm.at[pl.ds(base, _BPW)], uidx_v)
    pltpu.sync_copy(mid_hbm.at[pl.ds(base, _BPW)], midx_v)

    def to_mrow(x):
        return jnp.bitwise_or(
            lax.shift_left(lax.shift_right_logical(x, 14), 13),
            jnp.bitwise_and(x, _BTC - 1))

    def halve(i, carry):
        sl = pl.ds(i * 16, 16)
        uidx_v[sl] = to_mrow(uidx_v[sl])
        midx_v[sl] = to_mrow(midx_v[sl])
        return carry

    lax.fori_loop(0, _BPW // 16, halve, 0)

    for rnd in range(_BPW // _R):
        hs = []
        for ci in range(_R // _CH):
            sl_src = pl.ds(rnd * _R + ci * _CH, _CH)
            sl_dst = pl.ds(ci * _CH, _CH)
            hs.append(pltpu.async_copy(
                u2_hbm.at[uidx_v.at[sl_src]], gu_v.at[sl_dst], sem))
            hs.append(pltpu.async_copy(
                m2_hbm.at[midx_v.at[sl_src]], gm_v.at[sl_dst], sem))
        for h in hs:
            h.wait()
        pltpu.sync_copy(gu_v, gu_out.at[pl.ds(base + rnd * _R, _R)])
        pltpu.sync_copy(gm_v, gm_out.at[pl.ds(base + rnd * _R, _R)])


@functools.cache
def _make_gather():
    return pl.kernel(
        _gather_body,
        mesh=plsc.VectorSubcoreMesh(core_axis_name="c", subcore_axis_name="s"),
        out_type=[
            jax.ShapeDtypeStruct((B, 2 * D), jnp.float32),
            jax.ShapeDtypeStruct((B, 2 * D), jnp.float32),
        ],
        scratch_types=[
            pltpu.VMEM((_BPW,), jnp.int32),
            pltpu.VMEM((_BPW,), jnp.int32),
            pltpu.VMEM((_R, 2 * D), jnp.float32),
            pltpu.VMEM((_R, 2 * D), jnp.float32),
            pltpu.SemaphoreType.DMA,
        ],
        compiler_params=pltpu.CompilerParams(use_tc_tiling_on_sc=True),
    )


_BTC = 8192


def _tr_body(x1_ref, x2_ref, o_ref):
    o_ref[...] = jnp.concatenate([x1_ref[...].T, x2_ref[...].T], axis=1)


@functools.cache
def _make_tr(n):
    grid = (n + 2 * _BTC - 1) // (2 * _BTC)
    nblk = (n + _BTC - 1) // _BTC
    return pl.pallas_call(
        _tr_body,
        grid=(grid,),
        in_specs=[
            pl.BlockSpec((D, _BTC), lambda i: (0, jnp.minimum(2 * i, nblk - 1))),
            pl.BlockSpec((D, _BTC),
                         lambda i: (0, jnp.minimum(2 * i + 1, nblk - 1))),
        ],
        out_specs=pl.BlockSpec((_BTC, 2 * D), lambda i: (i, 0)),
        out_shape=jax.ShapeDtypeStruct((grid * _BTC, 2 * D), jnp.float32),
        compiler_params=pltpu.CompilerParams(
            dimension_semantics=("arbitrary",),
        ),
    )


def _mlp_body(gu_ref, gm_ref, uid_ref, mid_ref, w1u_ref, w1m_ref, b1_ref,
              w2_ref, b2_ref, w3_ref, b3_ref, out_ref):
    pu = lax.bitwise_and(lax.shift_right_logical(uid_ref[...], 13), 1) == 1
    pm = lax.bitwise_and(lax.shift_right_logical(mid_ref[...], 13), 1) == 1
    u_ref = jnp.where(pu, gu_ref[:, D:], gu_ref[:, :D])
    m_ref = jnp.where(pm, gm_ref[:, D:], gm_ref[:, :D])
    h1 = jnp.dot(u_ref, w1u_ref[...], preferred_element_type=jnp.float32)
    h1 = h1 + jnp.dot(m_ref, w1m_ref[...], preferred_element_type=jnp.float32)
    h1 = jnp.maximum(h1 + b1_ref[...], 0.0)
    h2 = jnp.dot(h1, w2_ref[...], preferred_element_type=jnp.float32)
    h2 = jnp.maximum(h2 + b2_ref[...], 0.0)
    out_ref[...] = jnp.sum(h2 * w3_ref[...], axis=1, keepdims=True) + b3_ref[...]


@functools.cache
def _make_mlp():
    return pl.pallas_call(
        _mlp_body,
        grid=(B // _BT,),
        in_specs=[
            pl.BlockSpec((_BT, 2 * D), lambda i: (i, 0)),
            pl.BlockSpec((_BT, 2 * D), lambda i: (i, 0)),
            pl.BlockSpec((_BT, 1), lambda i: (i, 0)),
            pl.BlockSpec((_BT, 1), lambda i: (i, 0)),
            pl.BlockSpec((D, H1), lambda i: (0, 0)),
            pl.BlockSpec((D, H1), lambda i: (0, 0)),
            pl.BlockSpec((1, H1), lambda i: (0, 0)),
            pl.BlockSpec((H1, H2), lambda i: (0, 0)),
            pl.BlockSpec((1, H2), lambda i: (0, 0)),
            pl.BlockSpec((1, H2), lambda i: (0, 0)),
            pl.BlockSpec((1, 1), lambda i: (0, 0)),
        ],
        out_specs=pl.BlockSpec((_BT, 1), lambda i: (i, 0)),
        out_shape=jax.ShapeDtypeStruct((B, 1), jnp.float32),
        compiler_params=pltpu.CompilerParams(
            dimension_semantics=("arbitrary",),
        ),
    )


def kernel(user_id, movie_id, user_emb, movie_emb, W1, b1, W2, b2, W3, b3):
    uid = user_id.astype(jnp.int32)
    mid = movie_id.astype(jnp.int32)
    ut = user_emb.T
    mt = movie_emb.T
    u2 = _make_tr(user_emb.shape[0])(ut, ut)
    m2 = _make_tr(movie_emb.shape[0])(mt, mt)
    gu, gm = _make_gather()(uid, mid, u2, m2)
    return _make_mlp()(
        gu, gm, uid.reshape(B, 1), mid.reshape(B, 1),
        W1[:, :D].T, W1[:, D:].T, b1.reshape(1, H1),
        W2.T, b2.reshape(1, H2),
        W3.reshape(1, H2), b3.reshape(1, 1),
    )


# speedup vs baseline: 2.4821x; 1.1128x over previous
import functools

import jax
import jax.numpy as jnp
from jax import lax
from jax.experimental import pallas as pl
from jax.experimental.pallas import tpu as pltpu
from jax.experimental.pallas import tpu_sc as plsc

B = 16384
D = 64
H1 = 128
H2 = 64

_NC = 2
_NS = 16
_NW = _NC * _NS
_BPW = B // _NW
_CH = 128
_NCH = _BPW // _CH

_BT = 2048


_W = 16
_R = 256


def _gather_body(uid_hbm, mid_hbm, u2_hbm, m2_hbm, gu_out, gm_out,
                 uidx_v, midx_v, gu_v, gm_v, sem):
    wid = lax.axis_index("s") * _NC + lax.axis_index("c")
    base = wid * _BPW
    pltpu.sync_copy(uid_hbm.at[pl.ds(base, _BPW)], uidx_v)
    pltpu.sync_copy(mid_hbm.at[pl.ds(base, _BPW)], midx_v)

    def to_mrow(x):
        return jnp.bitwise_or(
            lax.shift_left(lax.shift_right_logical(x, 14), 13),
            jnp.bitwise_and(x, _BTC - 1))

    def halve(i, carry):
        sl = pl.ds(i * 16, 16)
        uidx_v[sl] = to_mrow(uidx_v[sl])
        midx_v[sl] = to_mrow(midx_v[sl])
        return carry

    lax.fori_loop(0, _BPW // 16, halve, 0)

    for rnd in range(_BPW // _R):
        hs = []
        for ci in range(_R // _CH):
            sl_src = pl.ds(rnd * _R + ci * _CH, _CH)
            sl_dst = pl.ds(ci * _CH, _CH)
            hs.append(pltpu.async_copy(
                u2_hbm.at[uidx_v.at[sl_src]], gu_v.at[sl_dst], sem))
            hs.append(pltpu.async_copy(
                m2_hbm.at[midx_v.at[sl_src]], gm_v.at[sl_dst], sem))
        for h in hs:
            h.wait()
        pltpu.sync_copy(gu_v, gu_out.at[pl.ds(base + rnd * _R, _R)])
        pltpu.sync_copy(gm_v, gm_out.at[pl.ds(base + rnd * _R, _R)])


@functools.cache
def _make_gather():
    return pl.kernel(
        _gather_body,
        mesh=plsc.VectorSubcoreMesh(core_axis_name="c", subcore_axis_name="s"),
        out_type=[
            jax.ShapeDtypeStruct((B, 2 * D), jnp.float32),
            jax.ShapeDtypeStruct((B, 2 * D), jnp.float32),
        ],
        scratch_types=[
            pltpu.VMEM((_BPW,), jnp.int32),
            pltpu.VMEM((_BPW,), jnp.int32),
            pltpu.VMEM((_R, 2 * D), jnp.float32),
            pltpu.VMEM((_R, 2 * D), jnp.float32),
            pltpu.SemaphoreType.DMA,
        ],
        compiler_params=pltpu.CompilerParams(use_tc_tiling_on_sc=True),
    )


_BTC = 8192


def _tr_body(x1_ref, x2_ref, o_ref):
    r = lax.broadcasted_iota(jnp.int32, (D, 2 * D), 0)
    c = lax.broadcasted_iota(jnp.int32, (D, 2 * D), 1)
    e1 = jnp.where(r == c, 1.0, 0.0)
    e2 = jnp.where(r + D == c, 1.0, 0.0)
    dn = (((0,), (0,)), ((), ()))
    o_ref[...] = (
        lax.dot_general(x1_ref[...], e1, dn, preferred_element_type=jnp.float32)
        + lax.dot_general(x2_ref[...], e2, dn, preferred_element_type=jnp.float32)
    )


@functools.cache
def _make_tr(n):
    grid = (n + 2 * _BTC - 1) // (2 * _BTC)
    nblk = (n + _BTC - 1) // _BTC
    return pl.pallas_call(
        _tr_body,
        grid=(grid,),
        in_specs=[
            pl.BlockSpec((D, _BTC), lambda i: (0, jnp.minimum(2 * i, nblk - 1))),
            pl.BlockSpec((D, _BTC),
                         lambda i: (0, jnp.minimum(2 * i + 1, nblk - 1))),
        ],
        out_specs=pl.BlockSpec((_BTC, 2 * D), lambda i: (i, 0)),
        out_shape=jax.ShapeDtypeStruct((grid * _BTC, 2 * D), jnp.float32),
        compiler_params=pltpu.CompilerParams(
            dimension_semantics=("arbitrary",),
        ),
    )


def _mlp_body(gu_ref, gm_ref, uid_ref, mid_ref, w1u_ref, w1m_ref, b1_ref,
              w2_ref, b2_ref, w3_ref, b3_ref, out_ref):
    pu = lax.bitwise_and(lax.shift_right_logical(uid_ref[...], 13), 1) == 1
    pm = lax.bitwise_and(lax.shift_right_logical(mid_ref[...], 13), 1) == 1
    u_ref = jnp.where(pu, gu_ref[:, D:], gu_ref[:, :D])
    m_ref = jnp.where(pm, gm_ref[:, D:], gm_ref[:, :D])
    h1 = jnp.dot(u_ref, w1u_ref[...], preferred_element_type=jnp.float32)
    h1 = h1 + jnp.dot(m_ref, w1m_ref[...], preferred_element_type=jnp.float32)
    h1 = jnp.maximum(h1 + b1_ref[...], 0.0)
    h2 = jnp.dot(h1, w2_ref[...], preferred_element_type=jnp.float32)
    h2 = jnp.maximum(h2 + b2_ref[...], 0.0)
    out_ref[...] = jnp.sum(h2 * w3_ref[...], axis=1, keepdims=True) + b3_ref[...]


@functools.cache
def _make_mlp():
    return pl.pallas_call(
        _mlp_body,
        grid=(B // _BT,),
        in_specs=[
            pl.BlockSpec((_BT, 2 * D), lambda i: (i, 0)),
            pl.BlockSpec((_BT, 2 * D), lambda i: (i, 0)),
            pl.BlockSpec((_BT, 1), lambda i: (i, 0)),
            pl.BlockSpec((_BT, 1), lambda i: (i, 0)),
            pl.BlockSpec((D, H1), lambda i: (0, 0)),
            pl.BlockSpec((D, H1), lambda i: (0, 0)),
            pl.BlockSpec((1, H1), lambda i: (0, 0)),
            pl.BlockSpec((H1, H2), lambda i: (0, 0)),
            pl.BlockSpec((1, H2), lambda i: (0, 0)),
            pl.BlockSpec((1, H2), lambda i: (0, 0)),
            pl.BlockSpec((1, 1), lambda i: (0, 0)),
        ],
        out_specs=pl.BlockSpec((_BT, 1), lambda i: (i, 0)),
        out_shape=jax.ShapeDtypeStruct((B, 1), jnp.float32),
        compiler_params=pltpu.CompilerParams(
            dimension_semantics=("arbitrary",),
        ),
    )


def kernel(user_id, movie_id, user_emb, movie_emb, W1, b1, W2, b2, W3, b3):
    uid = user_id.astype(jnp.int32)
    mid = movie_id.astype(jnp.int32)
    ut = user_emb.T
    mt = movie_emb.T
    u2 = _make_tr(user_emb.shape[0])(ut, ut)
    m2 = _make_tr(movie_emb.shape[0])(mt, mt)
    gu, gm = _make_gather()(uid, mid, u2, m2)
    return _make_mlp()(
        gu, gm, uid.reshape(B, 1), mid.reshape(B, 1),
        W1[:, :D].T, W1[:, D:].T, b1.reshape(1, H1),
        W2.T, b2.reshape(1, H2),
        W3.reshape(1, H2), b3.reshape(1, 1),
    )


# speedup vs baseline: 2.6682x; 1.0750x over previous
import functools

import jax
import jax.numpy as jnp
from jax import lax
from jax.experimental import pallas as pl
from jax.experimental.pallas import tpu as pltpu
from jax.experimental.pallas import tpu_sc as plsc

B = 16384
D = 64
H1 = 128
H2 = 64

_NC = 2
_NS = 16
_NW = _NC * _NS
_BPW = B // _NW
_CH = 128
_NCH = _BPW // _CH

_BT = 2048


_W = 16
_R = 256


def _gather_body(uid_hbm, mid_hbm, u2_hbm, m2_hbm, gu_out, gm_out,
                 uidx_v, midx_v, gu_v, gm_v, sem):
    wid = lax.axis_index("s") * _NC + lax.axis_index("c")
    base = wid * _BPW
    pltpu.sync_copy(uid_hbm.at[pl.ds(base, _BPW)], uidx_v)
    pltpu.sync_copy(mid_hbm.at[pl.ds(base, _BPW)], midx_v)

    def to_mrow(x):
        return jnp.bitwise_or(
            lax.shift_left(lax.shift_right_logical(x, _LOG2B + 1), _LOG2B),
            jnp.bitwise_and(x, _BTC - 1))

    def halve(i, carry):
        sl = pl.ds(i * 16, 16)
        uidx_v[sl] = to_mrow(uidx_v[sl])
        midx_v[sl] = to_mrow(midx_v[sl])
        return carry

    lax.fori_loop(0, _BPW // 16, halve, 0)

    for rnd in range(_BPW // _R):
        hs = []
        for ci in range(_R // _CH):
            sl_src = pl.ds(rnd * _R + ci * _CH, _CH)
            sl_dst = pl.ds(ci * _CH, _CH)
            hs.append(pltpu.async_copy(
                u2_hbm.at[uidx_v.at[sl_src]], gu_v.at[sl_dst], sem))
            hs.append(pltpu.async_copy(
                m2_hbm.at[midx_v.at[sl_src]], gm_v.at[sl_dst], sem))
        for h in hs:
            h.wait()
        pltpu.sync_copy(gu_v, gu_out.at[pl.ds(base + rnd * _R, _R)])
        pltpu.sync_copy(gm_v, gm_out.at[pl.ds(base + rnd * _R, _R)])


@functools.cache
def _make_gather():
    return pl.kernel(
        _gather_body,
        mesh=plsc.VectorSubcoreMesh(core_axis_name="c", subcore_axis_name="s"),
        out_type=[
            jax.ShapeDtypeStruct((B, 2 * D), jnp.float32),
            jax.ShapeDtypeStruct((B, 2 * D), jnp.float32),
        ],
        scratch_types=[
            pltpu.VMEM((_BPW,), jnp.int32),
            pltpu.VMEM((_BPW,), jnp.int32),
            pltpu.VMEM((_R, 2 * D), jnp.float32),
            pltpu.VMEM((_R, 2 * D), jnp.float32),
            pltpu.SemaphoreType.DMA,
        ],
        compiler_params=pltpu.CompilerParams(use_tc_tiling_on_sc=True),
    )


_BTC = 16384
_LOG2B = _BTC.bit_length() - 1


def _tr_body(x1_ref, x2_ref, o_ref):
    r = lax.broadcasted_iota(jnp.int32, (D, 2 * D), 0)
    c = lax.broadcasted_iota(jnp.int32, (D, 2 * D), 1)
    e1 = jnp.where(r == c, 1.0, 0.0)
    e2 = jnp.where(r + D == c, 1.0, 0.0)
    dn = (((0,), (0,)), ((), ()))
    o_ref[...] = (
        lax.dot_general(x1_ref[...], e1, dn, preferred_element_type=jnp.float32)
        + lax.dot_general(x2_ref[...], e2, dn, preferred_element_type=jnp.float32)
    )


@functools.cache
def _make_tr(n):
    grid = (n + 2 * _BTC - 1) // (2 * _BTC)
    nblk = (n + _BTC - 1) // _BTC
    return pl.pallas_call(
        _tr_body,
        grid=(grid,),
        in_specs=[
            pl.BlockSpec((D, _BTC), lambda i: (0, jnp.minimum(2 * i, nblk - 1))),
            pl.BlockSpec((D, _BTC),
                         lambda i: (0, jnp.minimum(2 * i + 1, nblk - 1))),
        ],
        out_specs=pl.BlockSpec((_BTC, 2 * D), lambda i: (i, 0)),
        out_shape=jax.ShapeDtypeStruct((grid * _BTC, 2 * D), jnp.float32),
        compiler_params=pltpu.CompilerParams(
            dimension_semantics=("arbitrary",),
        ),
    )


def _mlp_body(gu_ref, gm_ref, uid_ref, mid_ref, w1u_ref, w1m_ref, b1_ref,
              w2_ref, b2_ref, w3_ref, b3_ref, out_ref):
    pu = lax.bitwise_and(lax.shift_right_logical(uid_ref[...], _LOG2B), 1) == 1
    pm = lax.bitwise_and(lax.shift_right_logical(mid_ref[...], _LOG2B), 1) == 1
    u_ref = jnp.where(pu, gu_ref[:, D:], gu_ref[:, :D])
    m_ref = jnp.where(pm, gm_ref[:, D:], gm_ref[:, :D])
    h1 = jnp.dot(u_ref, w1u_ref[...], preferred_element_type=jnp.float32)
    h1 = h1 + jnp.dot(m_ref, w1m_ref[...], preferred_element_type=jnp.float32)
    h1 = jnp.maximum(h1 + b1_ref[...], 0.0)
    h2 = jnp.dot(h1, w2_ref[...], preferred_element_type=jnp.float32)
    h2 = jnp.maximum(h2 + b2_ref[...], 0.0)
    out_ref[...] = jnp.sum(h2 * w3_ref[...], axis=1, keepdims=True) + b3_ref[...]


@functools.cache
def _make_mlp():
    return pl.pallas_call(
        _mlp_body,
        grid=(B // _BT,),
        in_specs=[
            pl.BlockSpec((_BT, 2 * D), lambda i: (i, 0)),
            pl.BlockSpec((_BT, 2 * D), lambda i: (i, 0)),
            pl.BlockSpec((_BT, 1), lambda i: (i, 0)),
            pl.BlockSpec((_BT, 1), lambda i: (i, 0)),
            pl.BlockSpec((D, H1), lambda i: (0, 0)),
            pl.BlockSpec((D, H1), lambda i: (0, 0)),
            pl.BlockSpec((1, H1), lambda i: (0, 0)),
            pl.BlockSpec((H1, H2), lambda i: (0, 0)),
            pl.BlockSpec((1, H2), lambda i: (0, 0)),
            pl.BlockSpec((1, H2), lambda i: (0, 0)),
            pl.BlockSpec((1, 1), lambda i: (0, 0)),
        ],
        out_specs=pl.BlockSpec((_BT, 1), lambda i: (i, 0)),
        out_shape=jax.ShapeDtypeStruct((B, 1), jnp.float32),
        compiler_params=pltpu.CompilerParams(
            dimension_semantics=("arbitrary",),
        ),
    )


def kernel(user_id, movie_id, user_emb, movie_emb, W1, b1, W2, b2, W3, b3):
    uid = user_id.astype(jnp.int32)
    mid = movie_id.astype(jnp.int32)
    ut = user_emb.T
    mt = movie_emb.T
    u2 = _make_tr(user_emb.shape[0])(ut, ut)
    m2 = _make_tr(movie_emb.shape[0])(mt, mt)
    gu, gm = _make_gather()(uid, mid, u2, m2)
    return _make_mlp()(
        gu, gm, uid.reshape(B, 1), mid.reshape(B, 1),
        W1[:, :D].T, W1[:, D:].T, b1.reshape(1, H1),
        W2.T, b2.reshape(1, H2),
        W3.reshape(1, H2), b3.reshape(1, 1),
    )
